# baseline (speedup 1.0000x reference)
import functools

import jax
import jax.numpy as jnp
from jax import lax
from jax.experimental import pallas as pl
from jax.experimental.pallas import tpu as pltpu
from jax.experimental.pallas import tpu_sc as plsc

N = 3831
NP = 3840
IN = 1024
HID = 64
OUT = 32
R = 4
E = 65536
W1COL = HID + 16

BN = 256
NB = NP // BN

NTILES = 32
EPT = E // NTILES
GRP = 128
G = EPT // GRP
SROWS = (R * NP) // 16
ZROWS = 120


def _mm1_body(x_ref, w_ref, root_ref, o_ref):
    xb = x_ref[...]
    for r in range(R):
        o_ref[r, :, 0:HID] = jnp.dot(xb, w_ref[r],
                                     preferred_element_type=jnp.float32)
    o_ref[R, :, 0:HID] = jnp.dot(xb, root_ref[...],
                                 preferred_element_type=jnp.float32)
    o_ref[:, :, HID:W1COL] = jnp.ones((R + 1, BN, W1COL - HID), jnp.float32)


def _mm1(x, W1, root1):
    return pl.pallas_call(
        _mm1_body,
        grid=(NB,),
        in_specs=[
            pl.BlockSpec((BN, IN), lambda i: (i, 0)),
            pl.BlockSpec((R, IN, HID), lambda i: (0, 0, 0)),
            pl.BlockSpec((IN, HID), lambda i: (0, 0)),
        ],
        out_specs=pl.BlockSpec((R + 1, BN, W1COL), lambda i: (0, i, 0)),
        out_shape=jax.ShapeDtypeStruct((R + 1, NP, W1COL), jnp.float32),
    )(x, W1, root1)


@functools.cache
def _make_scatter(width, pad128=False):
    out_w = 128 if pad128 else width
    mesh = plsc.VectorSubcoreMesh(core_axis_name="c", subcore_axis_name="s")
    scratch = [
        pltpu.VMEM_SHARED((R * NP, width), jnp.float32),
        pltpu.VMEM((G, GRP), jnp.int32),
        pltpu.VMEM((G, GRP), jnp.int32),
        pltpu.VMEM((GRP, width), jnp.float32),
        pltpu.VMEM((GRP, width), jnp.float32),
        pltpu.VMEM((GRP, width), jnp.float32),
    ]
    if pad128:
        scratch.append(pltpu.VMEM((GRP, 128), jnp.float32))
    scratch += [pltpu.SemaphoreType.DMA, pltpu.SemaphoreType.DMA]

    @functools.partial(
        pl.kernel,
        mesh=mesh,
        compiler_params=pltpu.CompilerParams(use_tc_tiling_on_sc=False),
        out_type=jax.ShapeDtypeStruct((NTILES * SROWS, out_w), jnp.float32),
        scratch_types=scratch,
    )
    def scat(table, gidx, sidx, out, acc, gi_v, si_v,
             rows_a, rows_b, rows_c, *rest):
        if pad128:
            vb128, gsem, ssem = rest
        else:
            gsem, ssem = rest
        c = lax.axis_index("c")
        s = lax.axis_index("s")
        wid = c * 16 + s

        def zfill(i, carry):
            for k in range(width // 16):
                rows_c[i, pl.ds(k * 16, 16)] = jnp.zeros((16,), jnp.float32)
            return carry

        lax.fori_loop(0, ZROWS, zfill, 0)
        for z in range(SROWS // ZROWS):
            pltpu.sync_copy(rows_c.at[pl.ds(0, ZROWS)],
                            acc.at[pl.ds(s * SROWS + z * ZROWS, ZROWS)])
        pltpu.sync_copy(gidx.at[pl.ds(wid * G, G)], gi_v)
        pltpu.sync_copy(sidx.at[pl.ds(wid * G, G)], si_v)
        plsc.subcore_barrier()

        bufs = (rows_a, rows_b, rows_c)
        gcp = [None, None, None]
        scp = [None, None, None]
        gcp[0] = pltpu.async_copy(table.at[gi_v.at[0]], bufs[0], gsem)
        gcp[1] = pltpu.async_copy(table.at[gi_v.at[1]], bufs[1], gsem)
        for j in range(G):
            nj = j + 2
            if nj < G:
                if scp[nj % 3] is not None:
                    scp[nj % 3].wait()
                gcp[nj % 3] = pltpu.async_copy(
                    table.at[gi_v.at[nj]], bufs[nj % 3], gsem)
            gcp[j % 3].wait()
            scp[j % 3] = pltpu.async_copy(
                bufs[j % 3], acc.at[si_v.at[j]], ssem, add=True)
        scp[(G - 1) % 3].wait()
        scp[(G - 2) % 3].wait()
        scp[(G - 3) % 3].wait()
        plsc.subcore_barrier()
        if not pad128:
            pltpu.sync_copy(acc.at[pl.ds(s * SROWS, SROWS)],
                            out.at[pl.ds(wid * SROWS, SROWS)])
        else:
            def rep(i, carry):
                for k in range(width // 16):
                    vb128[i, pl.ds(k * 16, 16)] = rows_a[i, pl.ds(k * 16, 16)]
                return carry

            for z in range(SROWS // ZROWS):
                pltpu.sync_copy(acc.at[pl.ds(s * SROWS + z * ZROWS, ZROWS)],
                                rows_a.at[pl.ds(0, ZROWS)])
                lax.fori_loop(0, ZROWS, rep, 0)
                pltpu.sync_copy(vb128.at[pl.ds(0, ZROWS)],
                                out.at[pl.ds(wid * SROWS + z * ZROWS, ZROWS)])

    return scat


def _comb1_body(p_ref, root_ref, b1_ref, w2_ref, root2_ref, t2_ref, rcp_ref):
    p = p_ref[0] + p_ref[1]
    cnt = jnp.maximum(p[:, :, HID:HID + 1], 1.0)
    rcp = 1.0 / cnt
    rcp = rcp * (2.0 - cnt * rcp)
    h = jnp.sum(p[:, :, 0:HID] * rcp, axis=0)
    h = h + root_ref[0, :, 0:HID] + b1_ref[0]
    h = jnp.maximum(h, 0.0)
    for r in range(R):
        t2_ref[r] = jnp.dot(h, w2_ref[r], preferred_element_type=jnp.float32)
    t2_ref[R] = jnp.dot(h, root2_ref[...], preferred_element_type=jnp.float32)
    rcp_ref[...] = jnp.broadcast_to(rcp, (R, BN, OUT))


def _comb1(psum1, table1, b1, W2, root2):
    return pl.pallas_call(
        _comb1_body,
        grid=(NB,),
        in_specs=[
            pl.BlockSpec((2, R, BN, 128), lambda i: (0, 0, i, 0)),
            pl.BlockSpec((1, BN, W1COL), lambda i: (R, i, 0)),
            pl.BlockSpec((1, HID), lambda i: (0, 0)),
            pl.BlockSpec((R, HID, OUT), lambda i: (0, 0, 0)),
            pl.BlockSpec((HID, OUT), lambda i: (0, 0)),
        ],
        out_specs=[
            pl.BlockSpec((R + 1, BN, OUT), lambda i: (0, i, 0)),
            pl.BlockSpec((R, BN, OUT), lambda i: (0, i, 0)),
        ],
        out_shape=[
            jax.ShapeDtypeStruct((R + 1, NP, OUT), jnp.float32),
            jax.ShapeDtypeStruct((R, NP, OUT), jnp.float32),
        ],
    )(psum1, table1, b1, W2, root2)


def _comb2_body(p_ref, rcp_ref, root_ref, b2_ref, o_ref):
    q = p_ref[0] + p_ref[1]
    acc = root_ref[0] + b2_ref[0]
    for r in range(R):
        acc = acc + q[:, r * OUT:(r + 1) * OUT] * rcp_ref[r]
    o_ref[...] = acc


def _comb2(psum2, rcpb, table2, b2):
    return pl.pallas_call(
        _comb2_body,
        grid=(NB,),
        in_specs=[
            pl.BlockSpec((2, BN, R * OUT), lambda i: (0, i, 0)),
            pl.BlockSpec((R, BN, OUT), lambda i: (0, i, 0)),
            pl.BlockSpec((1, BN, OUT), lambda i: (R, i, 0)),
            pl.BlockSpec((1, OUT), lambda i: (0, 0)),
        ],
        out_specs=pl.BlockSpec((BN, OUT), lambda i: (i, 0)),
        out_shape=jax.ShapeDtypeStruct((N, OUT), jnp.float32),
    )(psum2, rcpb, table2, b2)


def kernel(x, edge_index, edge_type, W1, root1, b1, W2, root2, b2):
    et = edge_type.astype(jnp.int32)
    src = edge_index[0].astype(jnp.int32)
    dst = edge_index[1].astype(jnp.int32)
    gidx = (et * NP + src).reshape(E // GRP, GRP)
    sidx = (et * NP + dst).reshape(E // GRP, GRP)
    sidx2 = (dst * R + et).reshape(E // GRP, GRP)

    table1 = _mm1(x, W1, root1)
    t1_flat = table1.reshape((R + 1) * NP, W1COL)
    p1 = _make_scatter(W1COL, True)(t1_flat, gidx, sidx)
    psum1 = p1.reshape(2, R, NP, 128)

    table2, rcpb = _comb1(psum1, table1, b1[None, :], W2, root2)
    t2_flat = table2.reshape((R + 1) * NP, OUT)
    p2 = _make_scatter(OUT)(t2_flat, gidx, sidx2)
    psum2 = p2.reshape(2, NP, R * OUT)

    return _comb2(psum2, rcpb, table2, b2[None, :])


# speedup vs baseline: 1.0469x; 1.0469x over previous
import functools

import jax
import jax.numpy as jnp
from jax import lax
from jax.experimental import pallas as pl
from jax.experimental.pallas import tpu as pltpu
from jax.experimental.pallas import tpu_sc as plsc

N = 3831
NP = 3840
IN = 1024
HID = 64
OUT = 32
R = 4
E = 65536
W1COL = HID + 16

BN = 256
NB = NP // BN

NTILES = 32
EPT = E // NTILES
GRP = 128
G = EPT // GRP
SROWS = (R * NP) // 16
ZROWS = 120
ZR2 = 60


def _mm1_body(x_ref, w_ref, root_ref, o_ref):
    xb = x_ref[...]
    for r in range(R):
        o_ref[r, :, 0:HID] = jnp.dot(xb, w_ref[r],
                                     preferred_element_type=jnp.float32)
    o_ref[R, :, 0:HID] = jnp.dot(xb, root_ref[...],
                                 preferred_element_type=jnp.float32)
    o_ref[:, :, HID:W1COL] = jnp.ones((R + 1, BN, W1COL - HID), jnp.float32)


def _mm1(x, W1, root1):
    return pl.pallas_call(
        _mm1_body,
        grid=(NB,),
        in_specs=[
            pl.BlockSpec((BN, IN), lambda i: (i, 0)),
            pl.BlockSpec((R, IN, HID), lambda i: (0, 0, 0)),
            pl.BlockSpec((IN, HID), lambda i: (0, 0)),
        ],
        out_specs=pl.BlockSpec((R + 1, BN, W1COL), lambda i: (0, i, 0)),
        out_shape=jax.ShapeDtypeStruct((R + 1, NP, W1COL), jnp.float32),
    )(x, W1, root1)


@functools.cache
def _make_scatter(width, pad128=False):
    out_w = 128 if pad128 else width
    mesh = plsc.VectorSubcoreMesh(core_axis_name="c", subcore_axis_name="s")
    scratch = [
        pltpu.VMEM_SHARED((R * NP, width), jnp.float32),
        pltpu.VMEM((G, GRP), jnp.int32),
        pltpu.VMEM((G, GRP), jnp.int32),
        pltpu.VMEM((GRP, width), jnp.float32),
        pltpu.VMEM((GRP, width), jnp.float32),
        pltpu.VMEM((GRP, width), jnp.float32),
    ]
    if pad128:
        scratch.append(pltpu.VMEM((2, ZR2, 128), jnp.float32))
    scratch += [pltpu.SemaphoreType.DMA, pltpu.SemaphoreType.DMA]

    @functools.partial(
        pl.kernel,
        mesh=mesh,
        compiler_params=pltpu.CompilerParams(use_tc_tiling_on_sc=False),
        out_type=jax.ShapeDtypeStruct((NTILES * SROWS, out_w), jnp.float32),
        scratch_types=scratch,
    )
    def scat(table, gidx, sidx, out, acc, gi_v, si_v,
             rows_a, rows_b, rows_c, *rest):
        if pad128:
            vb128, gsem, ssem = rest
        else:
            gsem, ssem = rest
        c = lax.axis_index("c")
        s = lax.axis_index("s")
        wid = c * 16 + s

        def zfill(i, carry):
            for k in range(width // 16):
                rows_c[i, pl.ds(k * 16, 16)] = jnp.zeros((16,), jnp.float32)
            return carry

        lax.fori_loop(0, ZROWS, zfill, 0)
        for z in range(SROWS // ZROWS):
            pltpu.sync_copy(rows_c.at[pl.ds(0, ZROWS)],
                            acc.at[pl.ds(s * SROWS + z * ZROWS, ZROWS)])
        pltpu.sync_copy(gidx.at[pl.ds(wid * G, G)], gi_v)
        pltpu.sync_copy(sidx.at[pl.ds(wid * G, G)], si_v)
        plsc.subcore_barrier()

        bufs = (rows_a, rows_b, rows_c)
        gcp = [None, None, None]
        scp = [None, None, None]
        gcp[0] = pltpu.async_copy(table.at[gi_v.at[0]], bufs[0], gsem)
        gcp[1] = pltpu.async_copy(table.at[gi_v.at[1]], bufs[1], gsem)
        for j in range(G):
            nj = j + 2
            if nj < G:
                if scp[nj % 3] is not None:
                    scp[nj % 3].wait()
                gcp[nj % 3] = pltpu.async_copy(
                    table.at[gi_v.at[nj]], bufs[nj % 3], gsem)
            gcp[j % 3].wait()
            scp[j % 3] = pltpu.async_copy(
                bufs[j % 3], acc.at[si_v.at[j]], ssem, add=True)
        scp[(G - 1) % 3].wait()
        scp[(G - 2) % 3].wait()
        scp[(G - 3) % 3].wait()
        plsc.subcore_barrier()
        if not pad128:
            pltpu.sync_copy(acc.at[pl.ds(s * SROWS, SROWS)],
                            out.at[pl.ds(wid * SROWS, SROWS)])
        else:
            NZ = SROWS // ZR2
            inb = (rows_a, rows_b)

            def rep(b):
                def body(i, carry):
                    for k in range(width // 16):
                        vb128[b, i, pl.ds(k * 16, 16)] = \
                            inb[b][i, pl.ds(k * 16, 16)]
                    return carry
                lax.fori_loop(0, ZR2, body, 0)

            icp = [None, None]
            ocp = [None, None]
            icp[0] = pltpu.async_copy(acc.at[pl.ds(s * SROWS, ZR2)],
                                      inb[0].at[pl.ds(0, ZR2)], gsem)
            for z in range(NZ):
                b = z % 2
                nb = (z + 1) % 2
                if z + 1 < NZ:
                    icp[nb] = pltpu.async_copy(
                        acc.at[pl.ds(s * SROWS + (z + 1) * ZR2, ZR2)],
                        inb[nb].at[pl.ds(0, ZR2)], gsem)
                icp[b].wait()
                if ocp[b] is not None:
                    ocp[b].wait()
                rep(b)
                ocp[b] = pltpu.async_copy(
                    vb128.at[b], out.at[pl.ds(wid * SROWS + z * ZR2, ZR2)],
                    ssem)
            ocp[(NZ - 1) % 2].wait()
            ocp[(NZ - 2) % 2].wait()

    return scat


def _comb1_body(p_ref, root_ref, b1_ref, w2_ref, root2_ref, t2_ref, rcp_ref):
    p = p_ref[0] + p_ref[1]
    cnt = jnp.maximum(p[:, :, HID:HID + 1], 1.0)
    rcp = 1.0 / cnt
    rcp = rcp * (2.0 - cnt * rcp)
    h = jnp.sum(p[:, :, 0:HID] * rcp, axis=0)
    h = h + root_ref[0, :, 0:HID] + b1_ref[0]
    h = jnp.maximum(h, 0.0)
    for r in range(R):
        t2_ref[r] = jnp.dot(h, w2_ref[r], preferred_element_type=jnp.float32)
    t2_ref[R] = jnp.dot(h, root2_ref[...], preferred_element_type=jnp.float32)
    rcp_ref[...] = jnp.broadcast_to(rcp, (R, BN, OUT))


def _comb1(psum1, table1, b1, W2, root2):
    return pl.pallas_call(
        _comb1_body,
        grid=(NB,),
        in_specs=[
            pl.BlockSpec((2, R, BN, 128), lambda i: (0, 0, i, 0)),
            pl.BlockSpec((1, BN, W1COL), lambda i: (R, i, 0)),
            pl.BlockSpec((1, HID), lambda i: (0, 0)),
            pl.BlockSpec((R, HID, OUT), lambda i: (0, 0, 0)),
            pl.BlockSpec((HID, OUT), lambda i: (0, 0)),
        ],
        out_specs=[
            pl.BlockSpec((R + 1, BN, OUT), lambda i: (0, i, 0)),
            pl.BlockSpec((R, BN, OUT), lambda i: (0, i, 0)),
        ],
        out_shape=[
            jax.ShapeDtypeStruct((R + 1, NP, OUT), jnp.float32),
            jax.ShapeDtypeStruct((R, NP, OUT), jnp.float32),
        ],
    )(psum1, table1, b1, W2, root2)


def _comb2_body(p_ref, rcp_ref, root_ref, b2_ref, o_ref):
    q = p_ref[0] + p_ref[1]
    acc = root_ref[0] + b2_ref[0]
    for r in range(R):
        acc = acc + q[:, r * OUT:(r + 1) * OUT] * rcp_ref[r]
    o_ref[...] = acc


def _comb2(psum2, rcpb, table2, b2):
    return pl.pallas_call(
        _comb2_body,
        grid=(NB,),
        in_specs=[
            pl.BlockSpec((2, BN, R * OUT), lambda i: (0, i, 0)),
            pl.BlockSpec((R, BN, OUT), lambda i: (0, i, 0)),
            pl.BlockSpec((1, BN, OUT), lambda i: (R, i, 0)),
            pl.BlockSpec((1, OUT), lambda i: (0, 0)),
        ],
        out_specs=pl.BlockSpec((BN, OUT), lambda i: (i, 0)),
        out_shape=jax.ShapeDtypeStruct((N, OUT), jnp.float32),
    )(psum2, rcpb, table2, b2)


def kernel(x, edge_index, edge_type, W1, root1, b1, W2, root2, b2):
    et = edge_type.astype(jnp.int32)
    src = edge_index[0].astype(jnp.int32)
    dst = edge_index[1].astype(jnp.int32)
    gidx = (et * NP + src).reshape(E // GRP, GRP)
    sidx = (et * NP + dst).reshape(E // GRP, GRP)
    sidx2 = (dst * R + et).reshape(E // GRP, GRP)

    table1 = _mm1(x, W1, root1)
    t1_flat = table1.reshape((R + 1) * NP, W1COL)
    p1 = _make_scatter(W1COL, True)(t1_flat, gidx, sidx)
    psum1 = p1.reshape(2, R, NP, 128)

    table2, rcpb = _comb1(psum1, table1, b1[None, :], W2, root2)
    t2_flat = table2.reshape((R + 1) * NP, OUT)
    p2 = _make_scatter(OUT)(t2_flat, gidx, sidx2)
    psum2 = p2.reshape(2, NP, R * OUT)

    return _comb2(psum2, rcpb, table2, b2[None, :])


# speedup vs baseline: 1.1191x; 1.0689x over previous
import functools

import jax
import jax.numpy as jnp
from jax import lax
from jax.experimental import pallas as pl
from jax.experimental.pallas import tpu as pltpu
from jax.experimental.pallas import tpu_sc as plsc

N = 3831
NP = 3840
IN = 1024
HID = 64
OUT = 32
R = 4
E = 65536
W1COL = HID + 16

BN = 384
NB = NP // BN

NTILES = 32
EPT = E // NTILES
GRP = 128
G = EPT // GRP
SROWS = (R * NP) // 16
ZROWS = 120
ZR2 = 60


def _mm1_body(x_ref, w_ref, root_ref, o_ref):
    xb = x_ref[...]
    for r in range(R):
        o_ref[r, :, 0:HID] = jnp.dot(xb, w_ref[r],
                                     preferred_element_type=jnp.float32)
    o_ref[R, :, 0:HID] = jnp.dot(xb, root_ref[...],
                                 preferred_element_type=jnp.float32)
    o_ref[:, :, HID:W1COL] = jnp.ones((R + 1, BN, W1COL - HID), jnp.float32)


def _mm1(x, W1, root1):
    return pl.pallas_call(
        _mm1_body,
        grid=(NB,),
        in_specs=[
            pl.BlockSpec((BN, IN), lambda i: (i, 0)),
            pl.BlockSpec((R, IN, HID), lambda i: (0, 0, 0)),
            pl.BlockSpec((IN, HID), lambda i: (0, 0)),
        ],
        out_specs=pl.BlockSpec((R + 1, BN, W1COL), lambda i: (0, i, 0)),
        out_shape=jax.ShapeDtypeStruct((R + 1, NP, W1COL), jnp.float32),
    )(x, W1, root1)


@functools.cache
def _make_scatter(width, pad128=False):
    out_w = 128 if pad128 else width
    mesh = plsc.VectorSubcoreMesh(core_axis_name="c", subcore_axis_name="s")
    scratch = [
        pltpu.VMEM_SHARED((R * NP, width), jnp.float32),
        pltpu.VMEM((G, GRP), jnp.int32),
        pltpu.VMEM((G, GRP), jnp.int32),
        pltpu.VMEM((GRP, width), jnp.float32),
        pltpu.VMEM((GRP, width), jnp.float32),
        pltpu.VMEM((GRP, width), jnp.float32),
    ]
    if pad128:
        scratch.append(pltpu.VMEM((2, ZR2, 128), jnp.float32))
    scratch += [pltpu.SemaphoreType.DMA, pltpu.SemaphoreType.DMA]

    @functools.partial(
        pl.kernel,
        mesh=mesh,
        compiler_params=pltpu.CompilerParams(use_tc_tiling_on_sc=False),
        out_type=jax.ShapeDtypeStruct((NTILES * SROWS, out_w), jnp.float32),
        scratch_types=scratch,
    )
    def scat(table, gidx, sidx, out, acc, gi_v, si_v,
             rows_a, rows_b, rows_c, *rest):
        if pad128:
            vb128, gsem, ssem = rest
        else:
            gsem, ssem = rest
        c = lax.axis_index("c")
        s = lax.axis_index("s")
        wid = c * 16 + s

        def zfill(i, carry):
            for k in range(width // 16):
                rows_c[i, pl.ds(k * 16, 16)] = jnp.zeros((16,), jnp.float32)
            return carry

        lax.fori_loop(0, ZROWS, zfill, 0)
        for z in range(SROWS // ZROWS):
            pltpu.sync_copy(rows_c.at[pl.ds(0, ZROWS)],
                            acc.at[pl.ds(s * SROWS + z * ZROWS, ZROWS)])
        pltpu.sync_copy(gidx.at[pl.ds(wid * G, G)], gi_v)
        pltpu.sync_copy(sidx.at[pl.ds(wid * G, G)], si_v)
        plsc.subcore_barrier()

        bufs = (rows_a, rows_b, rows_c)
        gcp = [None, None, None]
        scp = [None, None, None]
        gcp[0] = pltpu.async_copy(table.at[gi_v.at[0]], bufs[0], gsem)
        gcp[1] = pltpu.async_copy(table.at[gi_v.at[1]], bufs[1], gsem)
        for j in range(G):
            nj = j + 2
            if nj < G:
                if scp[nj % 3] is not None:
                    scp[nj % 3].wait()
                gcp[nj % 3] = pltpu.async_copy(
                    table.at[gi_v.at[nj]], bufs[nj % 3], gsem)
            gcp[j % 3].wait()
            scp[j % 3] = pltpu.async_copy(
                bufs[j % 3], acc.at[si_v.at[j]], ssem, add=True)
        scp[(G - 1) % 3].wait()
        scp[(G - 2) % 3].wait()
        scp[(G - 3) % 3].wait()
        plsc.subcore_barrier()
        if not pad128:
            pltpu.sync_copy(acc.at[pl.ds(s * SROWS, SROWS)],
                            out.at[pl.ds(wid * SROWS, SROWS)])
        else:
            NZ = SROWS // ZR2
            inb = (rows_a, rows_b)

            def rep(b):
                def body(i, carry):
                    for k in range(width // 16):
                        vb128[b, i, pl.ds(k * 16, 16)] = \
                            inb[b][i, pl.ds(k * 16, 16)]
                    return carry
                lax.fori_loop(0, ZR2, body, 0)

            icp = [None, None]
            ocp = [None, None]
            icp[0] = pltpu.async_copy(acc.at[pl.ds(s * SROWS, ZR2)],
                                      inb[0].at[pl.ds(0, ZR2)], gsem)
            for z in range(NZ):
                b = z % 2
                nb = (z + 1) % 2
                if z + 1 < NZ:
                    icp[nb] = pltpu.async_copy(
                        acc.at[pl.ds(s * SROWS + (z + 1) * ZR2, ZR2)],
                        inb[nb].at[pl.ds(0, ZR2)], gsem)
                icp[b].wait()
                if ocp[b] is not None:
                    ocp[b].wait()
                rep(b)
                ocp[b] = pltpu.async_copy(
                    vb128.at[b], out.at[pl.ds(wid * SROWS + z * ZR2, ZR2)],
                    ssem)
            ocp[(NZ - 1) % 2].wait()
            ocp[(NZ - 2) % 2].wait()

    return scat


def _comb1_body(p_ref, root_ref, b1_ref, w2_ref, root2_ref, t2_ref, rcp_ref):
    p = p_ref[0] + p_ref[1]
    cnt = jnp.maximum(p[:, :, HID:HID + 1], 1.0)
    rcp = 1.0 / cnt
    rcp = rcp * (2.0 - cnt * rcp)
    h = jnp.sum(p[:, :, 0:HID] * rcp, axis=0)
    h = h + root_ref[0, :, 0:HID] + b1_ref[0]
    h = jnp.maximum(h, 0.0)
    for r in range(R):
        t2_ref[r] = jnp.dot(h, w2_ref[r], preferred_element_type=jnp.float32)
    t2_ref[R] = jnp.dot(h, root2_ref[...], preferred_element_type=jnp.float32)
    rcp_ref[...] = jnp.broadcast_to(rcp, (R, BN, OUT))


def _comb1(psum1, table1, b1, W2, root2):
    return pl.pallas_call(
        _comb1_body,
        grid=(NB,),
        in_specs=[
            pl.BlockSpec((2, R, BN, 128), lambda i: (0, 0, i, 0)),
            pl.BlockSpec((1, BN, W1COL), lambda i: (R, i, 0)),
            pl.BlockSpec((1, HID), lambda i: (0, 0)),
            pl.BlockSpec((R, HID, OUT), lambda i: (0, 0, 0)),
            pl.BlockSpec((HID, OUT), lambda i: (0, 0)),
        ],
        out_specs=[
            pl.BlockSpec((R + 1, BN, OUT), lambda i: (0, i, 0)),
            pl.BlockSpec((R, BN, OUT), lambda i: (0, i, 0)),
        ],
        out_shape=[
            jax.ShapeDtypeStruct((R + 1, NP, OUT), jnp.float32),
            jax.ShapeDtypeStruct((R, NP, OUT), jnp.float32),
        ],
    )(psum1, table1, b1, W2, root2)


def _comb2_body(p_ref, rcp_ref, root_ref, b2_ref, o_ref):
    q = p_ref[0] + p_ref[1]
    acc = root_ref[0] + b2_ref[0]
    for r in range(R):
        acc = acc + q[:, r * OUT:(r + 1) * OUT] * rcp_ref[r]
    o_ref[...] = acc


def _comb2(psum2, rcpb, table2, b2):
    return pl.pallas_call(
        _comb2_body,
        grid=(NB,),
        in_specs=[
            pl.BlockSpec((2, BN, R * OUT), lambda i: (0, i, 0)),
            pl.BlockSpec((R, BN, OUT), lambda i: (0, i, 0)),
            pl.BlockSpec((1, BN, OUT), lambda i: (R, i, 0)),
            pl.BlockSpec((1, OUT), lambda i: (0, 0)),
        ],
        out_specs=pl.BlockSpec((BN, OUT), lambda i: (i, 0)),
        out_shape=jax.ShapeDtypeStruct((N, OUT), jnp.float32),
    )(psum2, rcpb, table2, b2)


def kernel(x, edge_index, edge_type, W1, root1, b1, W2, root2, b2):
    et = edge_type.astype(jnp.int32)
    src = edge_index[0].astype(jnp.int32)
    dst = edge_index[1].astype(jnp.int32)
    gidx = (et * NP + src).reshape(E // GRP, GRP)
    sidx = (et * NP + dst).reshape(E // GRP, GRP)
    sidx2 = (dst * R + et).reshape(E // GRP, GRP)

    table1 = _mm1(x, W1, root1)
    t1_flat = table1.reshape((R + 1) * NP, W1COL)
    p1 = _make_scatter(W1COL, True)(t1_flat, gidx, sidx)
    psum1 = p1.reshape(2, R, NP, 128)

    table2, rcpb = _comb1(psum1, table1, b1[None, :], W2, root2)
    t2_flat = table2.reshape((R + 1) * NP, OUT)
    p2 = _make_scatter(OUT)(t2_flat, gidx, sidx2)
    psum2 = p2.reshape(2, NP, R * OUT)

    return _comb2(psum2, rcpb, table2, b2[None, :])


# speedup vs baseline: 1.1702x; 1.0457x over previous
import functools

import jax
import jax.numpy as jnp
from jax import lax
from jax.experimental import pallas as pl
from jax.experimental.pallas import tpu as pltpu
from jax.experimental.pallas import tpu_sc as plsc

N = 3831
NP = 3840
IN = 1024
HID = 64
OUT = 32
R = 4
E = 65536
W1COL = HID + 16

BN = 768
NB = NP // BN

NTILES = 32
EPT = E // NTILES
GRP = 128
G = EPT // GRP
SROWS = (R * NP) // 16
ZROWS = 120
ZR2 = 60


def _mm1_body(x_ref, w_ref, root_ref, o_ref):
    xb = x_ref[...]
    for r in range(R):
        o_ref[r, :, 0:HID] = jnp.dot(xb, w_ref[r],
                                     preferred_element_type=jnp.float32)
    o_ref[R, :, 0:HID] = jnp.dot(xb, root_ref[...],
                                 preferred_element_type=jnp.float32)
    o_ref[:, :, HID:W1COL] = jnp.ones((R + 1, BN, W1COL - HID), jnp.float32)


def _mm1(x, W1, root1):
    return pl.pallas_call(
        _mm1_body,
        grid=(NB,),
        in_specs=[
            pl.BlockSpec((BN, IN), lambda i: (i, 0)),
            pl.BlockSpec((R, IN, HID), lambda i: (0, 0, 0)),
            pl.BlockSpec((IN, HID), lambda i: (0, 0)),
        ],
        out_specs=pl.BlockSpec((R + 1, BN, W1COL), lambda i: (0, i, 0)),
        out_shape=jax.ShapeDtypeStruct((R + 1, NP, W1COL), jnp.float32),
    )(x, W1, root1)


@functools.cache
def _make_scatter(width, pad128=False):
    out_w = 128 if pad128 else width
    mesh = plsc.VectorSubcoreMesh(core_axis_name="c", subcore_axis_name="s")
    scratch = [
        pltpu.VMEM_SHARED((R * NP, width), jnp.float32),
        pltpu.VMEM((G, GRP), jnp.int32),
        pltpu.VMEM((G, GRP), jnp.int32),
        pltpu.VMEM((GRP, width), jnp.float32),
        pltpu.VMEM((GRP, width), jnp.float32),
        pltpu.VMEM((GRP, width), jnp.float32),
    ]
    if pad128:
        scratch.append(pltpu.VMEM((2, ZR2, 128), jnp.float32))
    scratch += [pltpu.SemaphoreType.DMA, pltpu.SemaphoreType.DMA]

    @functools.partial(
        pl.kernel,
        mesh=mesh,
        compiler_params=pltpu.CompilerParams(use_tc_tiling_on_sc=False),
        out_type=jax.ShapeDtypeStruct((NTILES * SROWS, out_w), jnp.float32),
        scratch_types=scratch,
    )
    def scat(table, gidx, sidx, out, acc, gi_v, si_v,
             rows_a, rows_b, rows_c, *rest):
        if pad128:
            vb128, gsem, ssem = rest
        else:
            gsem, ssem = rest
        c = lax.axis_index("c")
        s = lax.axis_index("s")
        wid = c * 16 + s

        def zfill(i, carry):
            for k in range(width // 16):
                rows_c[i, pl.ds(k * 16, 16)] = jnp.zeros((16,), jnp.float32)
            return carry

        lax.fori_loop(0, ZROWS, zfill, 0)
        for z in range(SROWS // ZROWS):
            pltpu.sync_copy(rows_c.at[pl.ds(0, ZROWS)],
                            acc.at[pl.ds(s * SROWS + z * ZROWS, ZROWS)])
        pltpu.sync_copy(gidx.at[pl.ds(wid * G, G)], gi_v)
        pltpu.sync_copy(sidx.at[pl.ds(wid * G, G)], si_v)
        plsc.subcore_barrier()

        bufs = (rows_a, rows_b, rows_c)
        gcp = [None, None, None]
        scp = [None, None, None]
        gcp[0] = pltpu.async_copy(table.at[gi_v.at[0]], bufs[0], gsem)
        gcp[1] = pltpu.async_copy(table.at[gi_v.at[1]], bufs[1], gsem)
        for j in range(G):
            nj = j + 2
            if nj < G:
                if scp[nj % 3] is not None:
                    scp[nj % 3].wait()
                gcp[nj % 3] = pltpu.async_copy(
                    table.at[gi_v.at[nj]], bufs[nj % 3], gsem)
            gcp[j % 3].wait()
            scp[j % 3] = pltpu.async_copy(
                bufs[j % 3], acc.at[si_v.at[j]], ssem, add=True)
        scp[(G - 1) % 3].wait()
        scp[(G - 2) % 3].wait()
        scp[(G - 3) % 3].wait()
        plsc.subcore_barrier()
        if not pad128:
            pltpu.sync_copy(acc.at[pl.ds(s * SROWS, SROWS)],
                            out.at[pl.ds(wid * SROWS, SROWS)])
        else:
            NZ = SROWS // ZR2
            inb = (rows_a, rows_b)

            def rep(b):
                def body(i, carry):
                    for k in range(width // 16):
                        vb128[b, i, pl.ds(k * 16, 16)] = \
                            inb[b][i, pl.ds(k * 16, 16)]
                    return carry
                lax.fori_loop(0, ZR2, body, 0)

            icp = [None, None]
            ocp = [None, None]
            icp[0] = pltpu.async_copy(acc.at[pl.ds(s * SROWS, ZR2)],
                                      inb[0].at[pl.ds(0, ZR2)], gsem)
            for z in range(NZ):
                b = z % 2
                nb = (z + 1) % 2
                if z + 1 < NZ:
                    icp[nb] = pltpu.async_copy(
                        acc.at[pl.ds(s * SROWS + (z + 1) * ZR2, ZR2)],
                        inb[nb].at[pl.ds(0, ZR2)], gsem)
                icp[b].wait()
                if ocp[b] is not None:
                    ocp[b].wait()
                rep(b)
                ocp[b] = pltpu.async_copy(
                    vb128.at[b], out.at[pl.ds(wid * SROWS + z * ZR2, ZR2)],
                    ssem)
            ocp[(NZ - 1) % 2].wait()
            ocp[(NZ - 2) % 2].wait()

    return scat


def _comb1_body(p_ref, root_ref, b1_ref, w2_ref, root2_ref, t2_ref, rcp_ref):
    p = p_ref[0] + p_ref[1]
    cnt = jnp.maximum(p[:, :, HID:HID + 1], 1.0)
    rcp = 1.0 / cnt
    rcp = rcp * (2.0 - cnt * rcp)
    h = jnp.sum(p[:, :, 0:HID] * rcp, axis=0)
    h = h + root_ref[0, :, 0:HID] + b1_ref[0]
    h = jnp.maximum(h, 0.0)
    for r in range(R):
        t2_ref[r] = jnp.dot(h, w2_ref[r], preferred_element_type=jnp.float32)
    t2_ref[R] = jnp.dot(h, root2_ref[...], preferred_element_type=jnp.float32)
    rcp_ref[...] = jnp.broadcast_to(rcp, (R, BN, OUT))


def _comb1(psum1, table1, b1, W2, root2):
    return pl.pallas_call(
        _comb1_body,
        grid=(NB,),
        in_specs=[
            pl.BlockSpec((2, R, BN, 128), lambda i: (0, 0, i, 0)),
            pl.BlockSpec((1, BN, W1COL), lambda i: (R, i, 0)),
            pl.BlockSpec((1, HID), lambda i: (0, 0)),
            pl.BlockSpec((R, HID, OUT), lambda i: (0, 0, 0)),
            pl.BlockSpec((HID, OUT), lambda i: (0, 0)),
        ],
        out_specs=[
            pl.BlockSpec((R + 1, BN, OUT), lambda i: (0, i, 0)),
            pl.BlockSpec((R, BN, OUT), lambda i: (0, i, 0)),
        ],
        out_shape=[
            jax.ShapeDtypeStruct((R + 1, NP, OUT), jnp.float32),
            jax.ShapeDtypeStruct((R, NP, OUT), jnp.float32),
        ],
    )(psum1, table1, b1, W2, root2)


def _comb2_body(p_ref, rcp_ref, root_ref, b2_ref, o_ref):
    q = p_ref[0] + p_ref[1]
    acc = root_ref[0] + b2_ref[0]
    for r in range(R):
        acc = acc + q[:, r * OUT:(r + 1) * OUT] * rcp_ref[r]
    o_ref[...] = acc


def _comb2(psum2, rcpb, table2, b2):
    return pl.pallas_call(
        _comb2_body,
        grid=(NB,),
        in_specs=[
            pl.BlockSpec((2, BN, R * OUT), lambda i: (0, i, 0)),
            pl.BlockSpec((R, BN, OUT), lambda i: (0, i, 0)),
            pl.BlockSpec((1, BN, OUT), lambda i: (R, i, 0)),
            pl.BlockSpec((1, OUT), lambda i: (0, 0)),
        ],
        out_specs=pl.BlockSpec((BN, OUT), lambda i: (i, 0)),
        out_shape=jax.ShapeDtypeStruct((N, OUT), jnp.float32),
    )(psum2, rcpb, table2, b2)


def kernel(x, edge_index, edge_type, W1, root1, b1, W2, root2, b2):
    et = edge_type.astype(jnp.int32)
    src = edge_index[0].astype(jnp.int32)
    dst = edge_index[1].astype(jnp.int32)
    gidx = (et * NP + src).reshape(E // GRP, GRP)
    sidx = (et * NP + dst).reshape(E // GRP, GRP)
    sidx2 = (dst * R + et).reshape(E // GRP, GRP)

    table1 = _mm1(x, W1, root1)
    t1_flat = table1.reshape((R + 1) * NP, W1COL)
    p1 = _make_scatter(W1COL, True)(t1_flat, gidx, sidx)
    psum1 = p1.reshape(2, R, NP, 128)

    table2, rcpb = _comb1(psum1, table1, b1[None, :], W2, root2)
    t2_flat = table2.reshape((R + 1) * NP, OUT)
    p2 = _make_scatter(OUT)(t2_flat, gidx, sidx2)
    psum2 = p2.reshape(2, NP, R * OUT)

    return _comb2(psum2, rcpb, table2, b2[None, :])
